# Initial kernel scaffold; baseline (speedup 1.0000x reference)
import jax
import jax.numpy as jnp
from jax.experimental import pallas as pl


def kernel(x, edge_index, edge_weight, edge_label_index, explicit_weight, W1_rel, W1_root, b1, W2_rel, W2_root, b2, dec_W1, dec_b1, dec_W2, dec_b2):
    raise NotImplementedError("write your pallas kernel here")


# speedup vs baseline: 4.6572x; 4.6572x over previous
import functools

import jax
import jax.numpy as jnp
from jax import lax
from jax.experimental import pallas as pl
from jax.experimental.pallas import tpu as pltpu
from jax.experimental.pallas import tpu_sc as plsc

N = 10000
NP = 10240
D = 128
E = 320000
LBL = 100000

NC = 2
NS = 16
NW = NC * NS


_SEG_CH = 80
_ZROWS = 128


def _seg_body(with_cnt, src_tbl, src_hbm, dst_hbm, ew_hbm, out_hbm, cnt_hbm,
              src_v, dst_v, ew_v, ones_v, rows_v, zbuf_v, table_sh, cnt_sh,
              gsem):
    cid = lax.axis_index("c")
    sid = lax.axis_index("s")
    wid = sid * NC + cid
    epw = E // NW
    rows_pt = NP // NS

    def zfill(r, _):
        for j in range(D // 16):
            zbuf_v[r, pl.ds(j * 16, 16)] = jnp.zeros((16,), jnp.float32)
        return 0
    lax.fori_loop(0, _ZROWS, zfill, 0)
    for k in range(rows_pt // _ZROWS):
        pltpu.sync_copy(zbuf_v, table_sh.at[pl.ds(sid * rows_pt + k * _ZROWS, _ZROWS)])
    if with_cnt:
        def ofill(g, _):
            ones_v[pl.ds(g * 16, 16)] = jnp.full((16,), 1.0, jnp.float32)
            return 0
        lax.fori_loop(0, _SEG_CH // 16, ofill, 0)
        for k in range(rows_pt // _ZROWS):
            pltpu.sync_copy(zbuf_v.at[k, :],
                            cnt_sh.at[pl.ds(sid * rows_pt + k * _ZROWS, _ZROWS)])
    plsc.subcore_barrier()

    def chunk(ch, _):
        base = wid * epw + ch * _SEG_CH
        pltpu.sync_copy(src_hbm.at[pl.ds(base, _SEG_CH)], src_v)
        pltpu.sync_copy(dst_hbm.at[pl.ds(base, _SEG_CH)], dst_v)
        pltpu.sync_copy(ew_hbm.at[pl.ds(base, _SEG_CH)], ew_v)
        pltpu.async_copy(src_tbl.at[src_v], rows_v, gsem).wait()

        def scale(g, _):
            ew16 = ew_v[pl.ds(g * 16, 16)]
            for j in range(16):
                w = ew16[j]
                e = g * 16 + j
                for r in range(D // 16):
                    sl = pl.ds(r * 16, 16)
                    rows_v[e, sl] = rows_v[e, sl] * w
            return 0
        lax.fori_loop(0, _SEG_CH // 16, scale, 0)
        pltpu.sync_copy(rows_v, table_sh.at[dst_v], add=True)
        if with_cnt:
            pltpu.sync_copy(ones_v, cnt_sh.at[dst_v], add=True)
        return 0
    lax.fori_loop(0, epw // _SEG_CH, chunk, 0)

    plsc.subcore_barrier()
    pltpu.sync_copy(table_sh.at[pl.ds(sid * rows_pt, rows_pt)],
                    out_hbm.at[cid, pl.ds(sid * rows_pt, rows_pt)])
    if with_cnt:
        pltpu.sync_copy(cnt_sh.at[pl.ds(sid * rows_pt, rows_pt)],
                        cnt_hbm.at[pl.ds(cid * NP + sid * rows_pt, rows_pt)])


def _make_seg(with_cnt, mesh):
    outs = [jax.ShapeDtypeStruct((NC, NP, D), jnp.float32)]
    if with_cnt:
        outs.append(jax.ShapeDtypeStruct((NC * NP,), jnp.float32))
    def body(src_tbl, src_hbm, dst_hbm, ew_hbm, *rest):
        if with_cnt:
            out_hbm, cnt_hbm = rest[0], rest[1]
            scr = rest[2:]
        else:
            out_hbm, cnt_hbm = rest[0], None
            scr = rest[1:]
        _seg_body(with_cnt, src_tbl, src_hbm, dst_hbm, ew_hbm, out_hbm,
                  cnt_hbm, *scr)
    return pl.kernel(
        body,
        out_type=tuple(outs) if with_cnt else outs[0],
        mesh=mesh,
        scratch_types=[
            pltpu.VMEM((_SEG_CH,), jnp.int32),
            pltpu.VMEM((_SEG_CH,), jnp.int32),
            pltpu.VMEM((_SEG_CH,), jnp.float32),
            pltpu.VMEM((_SEG_CH,), jnp.float32),
            pltpu.VMEM((_SEG_CH, D), jnp.float32),
            pltpu.VMEM((_ZROWS, D), jnp.float32),
            pltpu.VMEM_SHARED((NP, D), jnp.float32),
            pltpu.VMEM_SHARED((NP,), jnp.float32),
            pltpu.SemaphoreType.DMA,
        ],
    )


_LP = 102400
_DEC_CH = 320


def _dec_body(zab_hbm, src_hbm, dst_hbm, ew_hbm, dp_hbm, out_hbm,
              sv, dv, ewv, ra_v, rb_v, out_v, dp_v, gsem):
    cid = lax.axis_index("c")
    sid = lax.axis_index("s")
    wid = sid * NC + cid
    epw = _LP // NW

    pltpu.sync_copy(dp_hbm, dp_v)
    base_iota = lax.iota(jnp.int32, 16)
    rots = [jnp.bitwise_and(base_iota + s, 15) for s in (8, 4, 2, 1)]
    lane_eq = [base_iota == j for j in range(16)]
    cv = [dp_v[pl.ds(16 * i, 16)] for i in range(4)]
    wv = [dp_v[pl.ds(64 + 16 * i, 16)] for i in range(4)]
    b2s = dp_v[pl.ds(128, 16)][0]

    def chunk(ch, _):
        base = wid * epw + ch * _DEC_CH
        pltpu.sync_copy(src_hbm.at[pl.ds(base, _DEC_CH)], sv)
        pltpu.sync_copy(dst_hbm.at[pl.ds(base, _DEC_CH)], dv)
        pltpu.sync_copy(ew_hbm.at[pl.ds(base, _DEC_CH)], ewv)
        ca = pltpu.async_copy(zab_hbm.at[sv], ra_v, gsem)
        cb = pltpu.async_copy(zab_hbm.at[dv], rb_v, gsem)
        ca.wait()
        cb.wait()

        def group(g, _):
            ew16 = ewv[pl.ds(g * 16, 16)]
            yvec = jnp.zeros((16,), jnp.float32)
            for j in range(16):
                e = g * 16 + j
                w = ew16[j]
                acc = None
                for r in range(4):
                    sl = pl.ds(r * 16, 16)
                    t = ra_v[e, sl] + rb_v[e, pl.ds(64 + r * 16, 16)] + w * cv[r]
                    t = jnp.maximum(t, 0.0)
                    tw = t * wv[r]
                    acc = tw if acc is None else acc + tw
                for rot in rots:
                    acc = acc + acc.at[rot].get(mode="promise_in_bounds")
                yvec = jnp.where(lane_eq[j], acc, yvec)
            out_v[pl.ds(g * 16, 16)] = yvec + b2s
            return 0
        lax.fori_loop(0, _DEC_CH // 16, group, 0)
        pltpu.sync_copy(out_v, out_hbm.at[pl.ds(base, _DEC_CH)])
        return 0
    lax.fori_loop(0, epw // _DEC_CH, chunk, 0)


def _make_dec(mesh):
    return pl.kernel(
        _dec_body,
        out_type=jax.ShapeDtypeStruct((_LP,), jnp.float32),
        mesh=mesh,
        scratch_types=[
            pltpu.VMEM((_DEC_CH,), jnp.int32),
            pltpu.VMEM((_DEC_CH,), jnp.int32),
            pltpu.VMEM((_DEC_CH,), jnp.float32),
            pltpu.VMEM((_DEC_CH, 128), jnp.float32),
            pltpu.VMEM((_DEC_CH, 128), jnp.float32),
            pltpu.VMEM((_DEC_CH,), jnp.float32),
            pltpu.VMEM((144,), jnp.float32),
            pltpu.SemaphoreType.DMA,
        ],
    )


_BR = 1280


def _tc1_body(s1_ref, c0_ref, c1_ref, x_ref, wrel_ref, wroot_ref, b_ref, o_ref):
    s = s1_ref[0] + s1_ref[1]
    cnt = c0_ref[...] + c1_ref[...]
    agg = s / jnp.maximum(cnt, 1.0)
    z = (jnp.dot(agg, wrel_ref[...], preferred_element_type=jnp.float32)
         + jnp.dot(x_ref[...], wroot_ref[...], preferred_element_type=jnp.float32)
         + b_ref[...])
    o_ref[...] = jnp.maximum(z, 0.0)


def _tc2_body(s2_ref, c0_ref, c1_ref, z1_ref, wrel_ref, wroot_ref, b_ref,
              a_ref, db1_ref, oa_ref):
    s2 = s2_ref[0] + s2_ref[1]
    cnt = c0_ref[...] + c1_ref[...]
    agg = s2 / jnp.maximum(cnt, 1.0)
    z2 = (jnp.dot(agg, wrel_ref[...], preferred_element_type=jnp.float32)
          + jnp.dot(z1_ref[...], wroot_ref[...], preferred_element_type=jnp.float32)
          + b_ref[...])
    oa_ref[...] = (jnp.dot(z2, a_ref[...], preferred_element_type=jnp.float32)
                   + db1_ref[...])


def _full(shape):
    return pl.BlockSpec(shape, lambda i: tuple(0 for _ in shape))


def kernel(x, edge_index, edge_weight, edge_label_index, explicit_weight,
           W1_rel, W1_root, b1, W2_rel, W2_root, b2,
           dec_W1, dec_b1, dec_W2, dec_b2):
    ei = edge_index.astype(jnp.int32)
    src, dst = ei[0], ei[1]
    eli = edge_label_index.astype(jnp.int32)
    pad = _LP - LBL
    padidx = jnp.arange(pad, dtype=jnp.int32) % N
    lsrc = jnp.concatenate([eli[0], padidx])
    ldst = jnp.concatenate([eli[1], padidx])
    ewl = jnp.concatenate([explicit_weight, jnp.zeros((pad,), jnp.float32)])

    xp = jnp.concatenate([x, jnp.zeros((NP - N, D), jnp.float32)])

    A = dec_W1[:128]
    Bm = dec_W1[128:256]
    cvec = dec_W1[256]
    dparams = jnp.concatenate(
        [cvec, dec_W2[:, 0], dec_b2, jnp.zeros((15,), jnp.float32)])

    mesh = plsc.VectorSubcoreMesh(core_axis_name="c", subcore_axis_name="s")

    S1, cnt = _make_seg(True, mesh)(xp, src, dst, edge_weight)
    c0 = cnt[:NP].reshape(NP, 1)
    c1 = cnt[NP:].reshape(NP, 1)

    tc1 = pl.pallas_call(
        _tc1_body,
        grid=(NP // _BR,),
        in_specs=[
            pl.BlockSpec((NC, _BR, D), lambda i: (0, i, 0)),
            pl.BlockSpec((_BR, 1), lambda i: (i, 0)),
            pl.BlockSpec((_BR, 1), lambda i: (i, 0)),
            pl.BlockSpec((_BR, D), lambda i: (i, 0)),
            _full((D, D)), _full((D, D)), _full((1, D)),
        ],
        out_specs=pl.BlockSpec((_BR, D), lambda i: (i, 0)),
        out_shape=jax.ShapeDtypeStruct((NP, D), jnp.float32),
    )
    z1 = tc1(S1, c0, c1, xp, W1_rel, W1_root, b1.reshape(1, D))

    S2 = _make_seg(False, mesh)(z1, src, dst, edge_weight)

    tc2 = pl.pallas_call(
        _tc2_body,
        grid=(NP // _BR,),
        in_specs=[
            pl.BlockSpec((NC, _BR, D), lambda i: (0, i, 0)),
            pl.BlockSpec((_BR, 1), lambda i: (i, 0)),
            pl.BlockSpec((_BR, 1), lambda i: (i, 0)),
            pl.BlockSpec((_BR, D), lambda i: (i, 0)),
            _full((D, D)), _full((D, D)), _full((1, D)),
            _full((D, D)), _full((1, D)),
        ],
        out_specs=pl.BlockSpec((_BR, D), lambda i: (i, 0)),
        out_shape=jax.ShapeDtypeStruct((NP, D), jnp.float32),
    )
    Wd = jnp.concatenate([A, Bm], axis=1)
    bd = jnp.concatenate([jnp.zeros((64,), jnp.float32), dec_b1]).reshape(1, D)
    zAB = tc2(S2, c0, c1, z1, W2_rel, W2_root, b2.reshape(1, D), Wd, bd)

    out = _make_dec(mesh)(zAB, lsrc, ldst, ewl, dparams)
    return out[:LBL]
